# Initial kernel scaffold; baseline (speedup 1.0000x reference)
import jax
import jax.numpy as jnp
from jax.experimental import pallas as pl


def kernel(x, edge_index, W_mlp, b_mlp, W_lin, b_lin):
    raise NotImplementedError("write your pallas kernel here")


# speedup vs baseline: 7.5071x; 7.5071x over previous
import functools

import jax
import jax.numpy as jnp
from jax import lax
from jax.experimental import pallas as pl
from jax.experimental.pallas import tpu as pltpu
from jax.experimental.pallas import tpu_sc as plsc

_NC = 2
_NS = 16
_G = 80


def _sc_segment_sum(x, src, dst, thr, zrows):
    n, d = x.shape
    e = src.shape[0]
    nw = _NC * _NS
    ec = e // nw
    nb = ec // _G
    arows = -(-(n + 1) // 128) * 128
    rpt = arows // _NS
    mesh = plsc.VectorSubcoreMesh(core_axis_name="c", subcore_axis_name="s")

    @functools.partial(
        pl.kernel,
        out_type=jax.ShapeDtypeStruct((_NC, arows, d), jnp.float32),
        mesh=mesh,
        scratch_types=[
            pltpu.VMEM_SHARED((arows, d), jnp.float32),
            pltpu.VMEM((ec,), jnp.int32),
            pltpu.VMEM((ec,), jnp.int32),
            pltpu.VMEM((n,), jnp.int32),
            pltpu.VMEM((_G, d), jnp.float32),
            pltpu.VMEM((_G,), jnp.int32),
            pltpu.SemaphoreType.DMA,
        ],
        compiler_params=pltpu.CompilerParams(needs_layout_passes=False),
    )
    def k(x_hbm, src_hbm, dst_hbm, thr_hbm, z_hbm, out_hbm,
          acc, srcb, dstb, thrb, rows, dsti, sem):
        c = lax.axis_index("c")
        s = lax.axis_index("s")
        w = c * _NS + s
        pltpu.sync_copy(z_hbm, acc.at[pl.ds(s * rpt, rpt)])
        e0 = w * ec
        pltpu.sync_copy(src_hbm.at[pl.ds(e0, ec)], srcb)
        pltpu.sync_copy(dst_hbm.at[pl.ds(e0, ec)], dstb)
        pltpu.sync_copy(thr_hbm, thrb)
        plsc.subcore_barrier()

        def batch(b, carry):
            gat = pltpu.async_copy(x_hbm.at[srcb.at[pl.ds(b * _G, _G)]],
                                   rows, sem)
            for j in range(_G // 16):
                off = b * _G + j * 16
                dv = dstb[pl.ds(off, 16)]
                tv = plsc.load_gather(thrb, [dv])
                ev = e0 + off + lax.iota(jnp.int32, 16)
                sel = ev < tv
                dsti[pl.ds(j * 16, 16)] = jnp.where(sel, dv, n)
            gat.wait()
            pltpu.sync_copy(rows, acc.at[dsti], add=True)
            return carry

        lax.fori_loop(0, nb, batch, 0)
        plsc.subcore_barrier()
        pltpu.sync_copy(acc.at[pl.ds(s * rpt, rpt)],
                        out_hbm.at[c, pl.ds(s * rpt, rpt)])

    return k(x, src, dst, thr, zrows)


def _tc_finish(x, s0, s1, cnt, w_lin, b_lin):
    n, d = x.shape
    blk = 400

    def body(x_ref, s0_ref, s1_ref, c_ref, w_ref, b_ref, o_ref):
        svec = s0_ref[...] + s1_ref[...]
        r = 0.5 / jnp.maximum(c_ref[...], 1.0)
        h = x_ref[...] + svec * r
        o_ref[...] = (
            jnp.dot(h, w_ref[...], preferred_element_type=jnp.float32)
            + b_ref[...]
        )

    return pl.pallas_call(
        body,
        grid=(n // blk,),
        in_specs=[
            pl.BlockSpec((blk, d), lambda i: (i, 0)),
            pl.BlockSpec((blk, d), lambda i: (i, 0)),
            pl.BlockSpec((blk, d), lambda i: (i, 0)),
            pl.BlockSpec((blk, 1), lambda i: (i, 0)),
            pl.BlockSpec((d, d), lambda i: (0, 0)),
            pl.BlockSpec((1, d), lambda i: (0, 0)),
        ],
        out_specs=pl.BlockSpec((blk, d), lambda i: (i, 0)),
        out_shape=jax.ShapeDtypeStruct((n, d), jnp.float32),
    )(x, s0, s1, cnt, w_lin, b_lin)


def kernel(x, edge_index, W_mlp, b_mlp, W_lin, b_lin):
    del W_mlp, b_mlp
    n, d = x.shape
    src = edge_index[0].astype(jnp.int32)
    dst = edge_index[1].astype(jnp.int32)

    ar = jnp.arange(n, dtype=jnp.int32)
    lo = jnp.searchsorted(dst, ar, side="left").astype(jnp.int32)
    hi = jnp.searchsorted(dst, ar, side="right").astype(jnp.int32)
    q = (hi - lo + 1) // 2
    thr = lo + q

    arows = -(-(n + 1) // 128) * 128
    zrows = jnp.zeros((arows // _NS, d), jnp.float32)
    spair = _sc_segment_sum(x, src, dst, thr, zrows)

    cnt = q.astype(jnp.float32)[:, None]
    return _tc_finish(x, spair[0, :n], spair[1, :n], cnt, W_lin,
                      b_lin[None, :])


# speedup vs baseline: 20.9344x; 2.7886x over previous
import functools

import jax
import jax.numpy as jnp
from jax import lax
from jax.experimental import pallas as pl
from jax.experimental.pallas import tpu as pltpu
from jax.experimental.pallas import tpu_sc as plsc

_NC = 2
_NS = 16
_G = 80


def _npad(n):
    return -(-(n + 1) // 256) * 256


def _sc_boundaries(dst, n):
    e = dst.shape[0]
    npad = _npad(n)
    eb = e // _NS
    gb = eb // 16
    npt = npad // _NS
    mesh = plsc.VectorSubcoreMesh(core_axis_name="c", subcore_axis_name="s")

    @functools.partial(
        pl.kernel,
        out_type=(
            jax.ShapeDtypeStruct((npad,), jnp.int32),
            jax.ShapeDtypeStruct((npad,), jnp.float32),
        ),
        mesh=mesh,
        scratch_types=[
            pltpu.VMEM_SHARED((_NS, npad), jnp.int32),
            pltpu.VMEM((eb + 32,), jnp.int32),
            pltpu.VMEM((npad,), jnp.int32),
            pltpu.VMEM((npad,), jnp.int32),
            pltpu.VMEM((_NS, npt), jnp.int32),
            pltpu.VMEM((npt,), jnp.int32),
            pltpu.VMEM((npt,), jnp.float32),
        ],
        compiler_params=pltpu.CompilerParams(needs_layout_passes=False),
    )
    def k(dst_hbm, thr_hbm, cnt_hbm, st_sh, dstb2, stl, enl, mrg, thrt, cntt):
        c = lax.axis_index("c")
        s = lax.axis_index("s")
        iota = lax.iota(jnp.int32, 16)
        neg1 = jnp.full((16,), -1, jnp.int32)

        b0 = s * eb
        pltpu.sync_copy(dst_hbm.at[pl.ds(b0, eb)], dstb2.at[pl.ds(16, eb)])

        @pl.when(s == 0)
        def _():
            dstb2[pl.ds(0, 16)] = neg1

        @pl.when(s > 0)
        def _():
            pltpu.sync_copy(dst_hbm.at[pl.ds(b0 - 16, 16)],
                            dstb2.at[pl.ds(0, 16)])

        @pl.when(s == _NS - 1)
        def _():
            dstb2[pl.ds(16 + eb, 16)] = neg1

        @pl.when(s < _NS - 1)
        def _():
            pltpu.sync_copy(dst_hbm.at[pl.ds(b0 + eb, 16)],
                            dstb2.at[pl.ds(16 + eb, 16)])

        def init_body(i, carry):
            stl[pl.ds(i * 16, 16)] = neg1
            enl[pl.ds(i * 16, 16)] = neg1
            return carry

        lax.fori_loop(0, npad // 16, init_body, 0)

        def bound_body(g, carry):
            o = g * 16
            cur = dstb2[pl.ds(o + 16, 16)]
            prev = plsc.load_gather(dstb2, [o + 15 + iota])
            nxt = plsc.load_gather(dstb2, [o + 17 + iota])
            ev = b0 + o + iota
            plsc.store_scatter(stl, [cur], ev, mask=cur != prev)
            plsc.store_scatter(enl, [cur], ev + 1, mask=cur != nxt)
            return carry

        lax.fori_loop(0, gb, bound_body, 0)

        pltpu.sync_copy(stl, st_sh.at[s])
        plsc.subcore_barrier()
        pltpu.sync_copy(st_sh.at[:, pl.ds(s * npt, npt)], mrg)

        def merge_start(g, carry):
            m = neg1
            for t in range(_NS):
                m = jnp.maximum(m, mrg[t, pl.ds(g * 16, 16)])
            thrt[pl.ds(g * 16, 16)] = m
            return carry

        lax.fori_loop(0, npt // 16, merge_start, 0)
        plsc.subcore_barrier()
        pltpu.sync_copy(enl, st_sh.at[s])
        plsc.subcore_barrier()
        pltpu.sync_copy(st_sh.at[:, pl.ds(s * npt, npt)], mrg)

        def merge_fin(g, carry):
            m = neg1
            for t in range(_NS):
                m = jnp.maximum(m, mrg[t, pl.ds(g * 16, 16)])
            start = thrt[pl.ds(g * 16, 16)]
            q = lax.shift_right_arithmetic(m - start + 1, 1)
            thrt[pl.ds(g * 16, 16)] = start + q
            cntt[pl.ds(g * 16, 16)] = q.astype(jnp.float32)
            return carry

        lax.fori_loop(0, npt // 16, merge_fin, 0)

        @pl.when(c == 0)
        def _():
            pltpu.sync_copy(thrt, thr_hbm.at[pl.ds(s * npt, npt)])
            pltpu.sync_copy(cntt, cnt_hbm.at[pl.ds(s * npt, npt)])

    return k(dst)


def _sc_aggregate(x, src, dst, thr, zrows):
    n, d = x.shape
    e = src.shape[0]
    npad = thr.shape[0]
    nw = _NC * _NS
    ec = e // nw
    nb = ec // _G
    rpt = npad // _NS
    mesh = plsc.VectorSubcoreMesh(core_axis_name="c", subcore_axis_name="s")

    @functools.partial(
        pl.kernel,
        out_type=jax.ShapeDtypeStruct((_NC, npad, d), jnp.float32),
        mesh=mesh,
        scratch_types=[
            pltpu.VMEM_SHARED((npad, d), jnp.float32),
            pltpu.VMEM((ec,), jnp.int32),
            pltpu.VMEM((ec,), jnp.int32),
            pltpu.VMEM((npad,), jnp.int32),
            pltpu.VMEM((_G, d), jnp.float32),
            pltpu.VMEM((_G,), jnp.int32),
            pltpu.SemaphoreType.DMA,
        ],
        compiler_params=pltpu.CompilerParams(needs_layout_passes=False),
    )
    def k(x_hbm, src_hbm, dst_hbm, thr_hbm, z_hbm, out_hbm,
          acc, srcb, dstb, thrb, rows, dsti, sem):
        c = lax.axis_index("c")
        s = lax.axis_index("s")
        w = c * _NS + s
        iota = lax.iota(jnp.int32, 16)
        pltpu.sync_copy(z_hbm, acc.at[pl.ds(s * rpt, rpt)])
        e0 = w * ec
        pltpu.sync_copy(src_hbm.at[pl.ds(e0, ec)], srcb)
        pltpu.sync_copy(dst_hbm.at[pl.ds(e0, ec)], dstb)
        pltpu.sync_copy(thr_hbm, thrb)
        plsc.subcore_barrier()

        def batch(b, carry):
            gat = pltpu.async_copy(x_hbm.at[srcb.at[pl.ds(b * _G, _G)]],
                                   rows, sem)
            for j in range(_G // 16):
                off = b * _G + j * 16
                dv = dstb[pl.ds(off, 16)]
                tv = plsc.load_gather(thrb, [dv])
                sel = (e0 + off + iota) < tv
                dsti[pl.ds(j * 16, 16)] = jnp.where(sel, dv, n)
            gat.wait()
            pltpu.sync_copy(rows, acc.at[dsti], add=True)
            return carry

        lax.fori_loop(0, nb, batch, 0)
        plsc.subcore_barrier()
        pltpu.sync_copy(acc.at[pl.ds(s * rpt, rpt)],
                        out_hbm.at[c, pl.ds(s * rpt, rpt)])

    return k(x, src, dst, thr, zrows)


def _tc_finish(x, s0, s1, cnt, w_lin, b_lin):
    n, d = x.shape
    blk = 400

    def body(x_ref, s0_ref, s1_ref, c_ref, w_ref, b_ref, o_ref):
        svec = s0_ref[...] + s1_ref[...]
        r = 0.5 / jnp.maximum(c_ref[...], 1.0)
        h = x_ref[...] + svec * r
        o_ref[...] = (
            jnp.dot(h, w_ref[...], preferred_element_type=jnp.float32)
            + b_ref[...]
        )

    return pl.pallas_call(
        body,
        grid=(n // blk,),
        in_specs=[
            pl.BlockSpec((blk, d), lambda i: (i, 0)),
            pl.BlockSpec((blk, d), lambda i: (i, 0)),
            pl.BlockSpec((blk, d), lambda i: (i, 0)),
            pl.BlockSpec((blk, 1), lambda i: (i, 0)),
            pl.BlockSpec((d, d), lambda i: (0, 0)),
            pl.BlockSpec((1, d), lambda i: (0, 0)),
        ],
        out_specs=pl.BlockSpec((blk, d), lambda i: (i, 0)),
        out_shape=jax.ShapeDtypeStruct((n, d), jnp.float32),
    )(x, s0, s1, cnt, w_lin, b_lin)


def kernel(x, edge_index, W_mlp, b_mlp, W_lin, b_lin):
    del W_mlp, b_mlp
    n, d = x.shape
    src = edge_index[0].astype(jnp.int32)
    dst = edge_index[1].astype(jnp.int32)

    thr, cnt = _sc_boundaries(dst, n)
    zrows = jnp.zeros((_npad(n) // _NS, d), jnp.float32)
    spair = _sc_aggregate(x, src, dst, thr, zrows)

    return _tc_finish(x, spair[0, :n], spair[1, :n], cnt[:n, None], W_lin,
                      b_lin[None, :])


# speedup vs baseline: 29.5375x; 1.4110x over previous
import functools

import jax
import jax.numpy as jnp
from jax import lax
from jax.experimental import pallas as pl
from jax.experimental.pallas import tpu as pltpu
from jax.experimental.pallas import tpu_sc as plsc

_NC = 2
_NS = 16
_G = 80


def _npad(n):
    return -(-(n + 1) // 256) * 256


def _sc_boundaries(dst, n):
    e = dst.shape[0]
    npad = _npad(n)
    eb = e // _NS
    gb = eb // 16
    npt = npad // _NS
    mesh = plsc.VectorSubcoreMesh(core_axis_name="c", subcore_axis_name="s")

    @functools.partial(
        pl.kernel,
        out_type=(
            jax.ShapeDtypeStruct((npad,), jnp.int32),
            jax.ShapeDtypeStruct((npad,), jnp.float32),
        ),
        mesh=mesh,
        scratch_types=[
            pltpu.VMEM_SHARED((_NS, npad), jnp.int32),
            pltpu.VMEM((eb + 32,), jnp.int32),
            pltpu.VMEM((npad,), jnp.int32),
            pltpu.VMEM((npad,), jnp.int32),
            pltpu.VMEM((_NS, npt), jnp.int32),
            pltpu.VMEM((npt,), jnp.int32),
            pltpu.VMEM((npt,), jnp.float32),
        ],
        compiler_params=pltpu.CompilerParams(needs_layout_passes=False),
    )
    def k(dst_hbm, thr_hbm, cnt_hbm, st_sh, dstb2, stl, enl, mrg, thrt, cntt):
        c = lax.axis_index("c")
        s = lax.axis_index("s")
        iota = lax.iota(jnp.int32, 16)
        neg1 = jnp.full((16,), -1, jnp.int32)

        b0 = s * eb
        pltpu.sync_copy(dst_hbm.at[pl.ds(b0, eb)], dstb2.at[pl.ds(16, eb)])

        @pl.when(s == 0)
        def _():
            dstb2[pl.ds(0, 16)] = neg1

        @pl.when(s > 0)
        def _():
            pltpu.sync_copy(dst_hbm.at[pl.ds(b0 - 16, 16)],
                            dstb2.at[pl.ds(0, 16)])

        @pl.when(s == _NS - 1)
        def _():
            dstb2[pl.ds(16 + eb, 16)] = neg1

        @pl.when(s < _NS - 1)
        def _():
            pltpu.sync_copy(dst_hbm.at[pl.ds(b0 + eb, 16)],
                            dstb2.at[pl.ds(16 + eb, 16)])

        def init_body(i, carry):
            stl[pl.ds(i * 16, 16)] = neg1
            enl[pl.ds(i * 16, 16)] = neg1
            return carry

        lax.fori_loop(0, npad // 16, init_body, 0)

        def bound_body(g, carry):
            o = g * 16
            cur = dstb2[pl.ds(o + 16, 16)]
            prev = plsc.load_gather(dstb2, [o + 15 + iota])
            nxt = plsc.load_gather(dstb2, [o + 17 + iota])
            ev = b0 + o + iota
            plsc.store_scatter(stl, [cur], ev, mask=cur != prev)
            plsc.store_scatter(enl, [cur], ev + 1, mask=cur != nxt)
            return carry

        lax.fori_loop(0, gb, bound_body, 0)

        pltpu.sync_copy(stl, st_sh.at[s])
        plsc.subcore_barrier()
        pltpu.sync_copy(st_sh.at[:, pl.ds(s * npt, npt)], mrg)

        def merge_start(g, carry):
            m = neg1
            for t in range(_NS):
                m = jnp.maximum(m, mrg[t, pl.ds(g * 16, 16)])
            thrt[pl.ds(g * 16, 16)] = m
            return carry

        lax.fori_loop(0, npt // 16, merge_start, 0)
        plsc.subcore_barrier()
        pltpu.sync_copy(enl, st_sh.at[s])
        plsc.subcore_barrier()
        pltpu.sync_copy(st_sh.at[:, pl.ds(s * npt, npt)], mrg)

        def merge_fin(g, carry):
            m = neg1
            for t in range(_NS):
                m = jnp.maximum(m, mrg[t, pl.ds(g * 16, 16)])
            start = thrt[pl.ds(g * 16, 16)]
            q = lax.shift_right_arithmetic(m - start + 1, 1)
            thrt[pl.ds(g * 16, 16)] = start + q
            cntt[pl.ds(g * 16, 16)] = q.astype(jnp.float32)
            return carry

        lax.fori_loop(0, npt // 16, merge_fin, 0)

        @pl.when(c == 0)
        def _():
            pltpu.sync_copy(thrt, thr_hbm.at[pl.ds(s * npt, npt)])
            pltpu.sync_copy(cntt, cnt_hbm.at[pl.ds(s * npt, npt)])

    return k(dst)


def _sc_aggregate(x, src, dst, thr, zrows):
    n, d = x.shape
    e = src.shape[0]
    npad = thr.shape[0]
    nw = _NC * _NS
    ec = e // nw
    nb = ec // _G
    rpt = npad // _NS
    mesh = plsc.VectorSubcoreMesh(core_axis_name="c", subcore_axis_name="s")

    @functools.partial(
        pl.kernel,
        out_type=jax.ShapeDtypeStruct((_NC, npad, d), jnp.float32),
        mesh=mesh,
        scratch_types=[
            pltpu.VMEM_SHARED((npad, d), jnp.float32),
            pltpu.VMEM((ec + 96,), jnp.int32),
            pltpu.VMEM((ec + 96,), jnp.int32),
            pltpu.VMEM((npad,), jnp.int32),
            pltpu.VMEM((_G, d), jnp.float32),
            pltpu.VMEM((_G,), jnp.int32),
            pltpu.SemaphoreType.DMA,
        ],
        compiler_params=pltpu.CompilerParams(needs_layout_passes=False),
    )
    def k(x_hbm, src_hbm, dst_hbm, thr_hbm, z_hbm, out_hbm,
          acc, srcb, dstb, thrb, rows, dsti, sem):
        c = lax.axis_index("c")
        s = lax.axis_index("s")
        w = c * _NS + s
        iota = lax.iota(jnp.int32, 16)
        pltpu.sync_copy(z_hbm, acc.at[pl.ds(s * rpt, rpt)])
        e0 = w * ec
        pltpu.sync_copy(src_hbm.at[pl.ds(e0, ec)], srcb.at[pl.ds(0, ec)])
        pltpu.sync_copy(dst_hbm.at[pl.ds(e0, ec)], dstb.at[pl.ds(0, ec)])
        pltpu.sync_copy(thr_hbm, thrb)
        plsc.subcore_barrier()

        def compact(g, wptr):
            off = g * 16
            dv = dstb[pl.ds(off, 16)]
            tv = plsc.load_gather(thrb, [dv])
            sel = (e0 + off + iota) < tv
            plsc.store_compressed(srcb.at[pl.ds(wptr, 16)],
                                  srcb[pl.ds(off, 16)], mask=sel)
            plsc.store_compressed(dstb.at[pl.ds(wptr, 16)], dv, mask=sel)
            return wptr + jnp.sum(sel.astype(jnp.int32))

        nsel = lax.fori_loop(0, ec // 16, compact, jnp.int32(0))
        nbs = lax.div(nsel + (_G - 1), _G)
        for j in range(_G // 16 + 1):
            plsc.store_scatter(dstb, [nsel + j * 16 + iota],
                               jnp.full((16,), n, jnp.int32))

        def batch(b, carry):
            gat = pltpu.async_copy(x_hbm.at[srcb.at[pl.ds(b * _G, _G)]],
                                   rows, sem)
            for j in range(_G // 16):
                dsti[pl.ds(j * 16, 16)] = dstb[pl.ds(b * _G + j * 16, 16)]
            gat.wait()
            pltpu.sync_copy(rows, acc.at[dsti], add=True)
            return carry

        lax.fori_loop(0, nbs, batch, 0)
        plsc.subcore_barrier()
        pltpu.sync_copy(acc.at[pl.ds(s * rpt, rpt)],
                        out_hbm.at[c, pl.ds(s * rpt, rpt)])

    return k(x, src, dst, thr, zrows)


def _tc_finish(x, s0, s1, cnt, w_lin, b_lin):
    n, d = x.shape
    blk = 400

    def body(x_ref, s0_ref, s1_ref, c_ref, w_ref, b_ref, o_ref):
        svec = s0_ref[...] + s1_ref[...]
        r = 0.5 / jnp.maximum(c_ref[...], 1.0)
        h = x_ref[...] + svec * r
        o_ref[...] = (
            jnp.dot(h, w_ref[...], preferred_element_type=jnp.float32)
            + b_ref[...]
        )

    return pl.pallas_call(
        body,
        grid=(n // blk,),
        in_specs=[
            pl.BlockSpec((blk, d), lambda i: (i, 0)),
            pl.BlockSpec((blk, d), lambda i: (i, 0)),
            pl.BlockSpec((blk, d), lambda i: (i, 0)),
            pl.BlockSpec((blk, 1), lambda i: (i, 0)),
            pl.BlockSpec((d, d), lambda i: (0, 0)),
            pl.BlockSpec((1, d), lambda i: (0, 0)),
        ],
        out_specs=pl.BlockSpec((blk, d), lambda i: (i, 0)),
        out_shape=jax.ShapeDtypeStruct((n, d), jnp.float32),
    )(x, s0, s1, cnt, w_lin, b_lin)


def kernel(x, edge_index, W_mlp, b_mlp, W_lin, b_lin):
    del W_mlp, b_mlp
    n, d = x.shape
    src = edge_index[0].astype(jnp.int32)
    dst = edge_index[1].astype(jnp.int32)

    thr, cnt = _sc_boundaries(dst, n)
    zrows = jnp.zeros((_npad(n) // _NS, d), jnp.float32)
    spair = _sc_aggregate(x, src, dst, thr, zrows)

    return _tc_finish(x, spair[0, :n], spair[1, :n], cnt[:n, None], W_lin,
                      b_lin[None, :])


# speedup vs baseline: 38.0636x; 1.2887x over previous
import functools

import jax
import jax.numpy as jnp
from jax import lax
from jax.experimental import pallas as pl
from jax.experimental.pallas import tpu as pltpu
from jax.experimental.pallas import tpu_sc as plsc

_NC = 2
_NS = 16
_G = 64


def _npad(n):
    return -(-(n + 1) // 256) * 256


def _sc_boundaries(dst, n):
    e = dst.shape[0]
    npad = _npad(n)
    eb = e // _NS
    gb = eb // 16
    npt = npad // _NS
    mesh = plsc.VectorSubcoreMesh(core_axis_name="c", subcore_axis_name="s")

    @functools.partial(
        pl.kernel,
        out_type=(
            jax.ShapeDtypeStruct((npad,), jnp.int32),
            jax.ShapeDtypeStruct((npad,), jnp.float32),
        ),
        mesh=mesh,
        scratch_types=[
            pltpu.VMEM_SHARED((_NS, npad), jnp.int32),
            pltpu.VMEM((eb + 32,), jnp.int32),
            pltpu.VMEM((npad,), jnp.int32),
            pltpu.VMEM((npad,), jnp.int32),
            pltpu.VMEM((_NS, npt), jnp.int32),
            pltpu.VMEM((npt,), jnp.int32),
            pltpu.VMEM((npt,), jnp.float32),
        ],
        compiler_params=pltpu.CompilerParams(needs_layout_passes=False),
    )
    def k(dst_hbm, thr_hbm, cnt_hbm, st_sh, dstb2, stl, enl, mrg, thrt, cntt):
        c = lax.axis_index("c")
        s = lax.axis_index("s")
        iota = lax.iota(jnp.int32, 16)
        neg1 = jnp.full((16,), -1, jnp.int32)

        b0 = s * eb
        pltpu.sync_copy(dst_hbm.at[pl.ds(b0, eb)], dstb2.at[pl.ds(16, eb)])

        @pl.when(s == 0)
        def _():
            dstb2[pl.ds(0, 16)] = neg1

        @pl.when(s > 0)
        def _():
            pltpu.sync_copy(dst_hbm.at[pl.ds(b0 - 16, 16)],
                            dstb2.at[pl.ds(0, 16)])

        @pl.when(s == _NS - 1)
        def _():
            dstb2[pl.ds(16 + eb, 16)] = neg1

        @pl.when(s < _NS - 1)
        def _():
            pltpu.sync_copy(dst_hbm.at[pl.ds(b0 + eb, 16)],
                            dstb2.at[pl.ds(16 + eb, 16)])

        def init_body(i, carry):
            stl[pl.ds(i * 16, 16)] = neg1
            enl[pl.ds(i * 16, 16)] = neg1
            return carry

        lax.fori_loop(0, npad // 16, init_body, 0)

        def bound_body(g, carry):
            o = g * 16
            cur = dstb2[pl.ds(o + 16, 16)]
            prev = plsc.load_gather(dstb2, [o + 15 + iota])
            nxt = plsc.load_gather(dstb2, [o + 17 + iota])
            ev = b0 + o + iota
            plsc.store_scatter(stl, [cur], ev, mask=cur != prev)
            plsc.store_scatter(enl, [cur], ev + 1, mask=cur != nxt)
            return carry

        lax.fori_loop(0, gb, bound_body, 0)

        pltpu.sync_copy(stl, st_sh.at[s])
        plsc.subcore_barrier()
        pltpu.sync_copy(st_sh.at[:, pl.ds(s * npt, npt)], mrg)

        def merge_start(g, carry):
            m = neg1
            for t in range(_NS):
                m = jnp.maximum(m, mrg[t, pl.ds(g * 16, 16)])
            thrt[pl.ds(g * 16, 16)] = m
            return carry

        lax.fori_loop(0, npt // 16, merge_start, 0)
        plsc.subcore_barrier()
        pltpu.sync_copy(enl, st_sh.at[s])
        plsc.subcore_barrier()
        pltpu.sync_copy(st_sh.at[:, pl.ds(s * npt, npt)], mrg)

        def merge_fin(g, carry):
            m = neg1
            for t in range(_NS):
                m = jnp.maximum(m, mrg[t, pl.ds(g * 16, 16)])
            start = thrt[pl.ds(g * 16, 16)]
            q = lax.shift_right_arithmetic(m - start + 1, 1)
            thrt[pl.ds(g * 16, 16)] = start + q
            cntt[pl.ds(g * 16, 16)] = q.astype(jnp.float32)
            return carry

        lax.fori_loop(0, npt // 16, merge_fin, 0)

        @pl.when(c == 0)
        def _():
            pltpu.sync_copy(thrt, thr_hbm.at[pl.ds(s * npt, npt)])
            pltpu.sync_copy(cntt, cnt_hbm.at[pl.ds(s * npt, npt)])

    return k(dst)


def _sc_aggregate(x, src, dst, thr, zrows):
    n, d = x.shape
    e = src.shape[0]
    npad = thr.shape[0]
    nw = _NC * _NS
    ec = e // nw
    rpt = npad // _NS
    mesh = plsc.VectorSubcoreMesh(core_axis_name="c", subcore_axis_name="s")

    @functools.partial(
        pl.kernel,
        out_type=jax.ShapeDtypeStruct((_NC, npad, d), jnp.float32),
        mesh=mesh,
        scratch_types=[
            pltpu.VMEM_SHARED((npad, d), jnp.float32),
            pltpu.VMEM((ec + 16 + _G,), jnp.int32),
            pltpu.VMEM((ec + 16 + _G,), jnp.int32),
            pltpu.VMEM((npad,), jnp.int32),
            pltpu.VMEM((_G, d), jnp.float32),
            pltpu.VMEM((_G, d), jnp.float32),
            pltpu.VMEM((_G,), jnp.int32),
            pltpu.VMEM((_G,), jnp.int32),
            pltpu.SemaphoreType.DMA,
            pltpu.SemaphoreType.DMA,
        ],
        compiler_params=pltpu.CompilerParams(needs_layout_passes=False),
    )
    def k(x_hbm, src_hbm, dst_hbm, thr_hbm, z_hbm, out_hbm,
          acc, srcb, dstb, thrb, rowsa, rowsb, dstia, dstib, sema, semb):
        c = lax.axis_index("c")
        s = lax.axis_index("s")
        w = c * _NS + s
        iota = lax.iota(jnp.int32, 16)
        pltpu.sync_copy(z_hbm, acc.at[pl.ds(s * rpt, rpt)])
        e0 = w * ec
        pltpu.sync_copy(src_hbm.at[pl.ds(e0, ec)], srcb.at[pl.ds(0, ec)])
        pltpu.sync_copy(dst_hbm.at[pl.ds(e0, ec)], dstb.at[pl.ds(0, ec)])
        pltpu.sync_copy(thr_hbm, thrb)
        plsc.subcore_barrier()

        def compact(g, wptr):
            off = g * 16
            dv = dstb[pl.ds(off, 16)]
            tv = plsc.load_gather(thrb, [dv])
            sel = (e0 + off + iota) < tv
            plsc.store_compressed(srcb.at[pl.ds(wptr, 16)],
                                  srcb[pl.ds(off, 16)], mask=sel)
            plsc.store_compressed(dstb.at[pl.ds(wptr, 16)], dv, mask=sel)
            return wptr + jnp.sum(sel.astype(jnp.int32))

        nsel = lax.fori_loop(0, ec // 16, compact, jnp.int32(0))
        nbs = lax.div(nsel + (_G - 1), _G)
        for j in range(_G // 16 + 1):
            plsc.store_scatter(dstb, [nsel + j * 16 + iota],
                               jnp.full((16,), n, jnp.int32))

        def fire(b, rows, sem):
            return pltpu.async_copy(x_hbm.at[srcb.at[pl.ds(b * _G, _G)]],
                                    rows, sem)

        def prep(b, dsti):
            for j in range(_G // 16):
                dsti[pl.ds(j * 16, 16)] = dstb[pl.ds(b * _G + j * 16, 16)]

        @pl.when(nbs > 0)
        def _():
            fire(0, rowsa, sema)

        def pipe(i, carry):
            b = 2 * i

            @pl.when(b + 1 < nbs)
            def _():
                fire(b + 1, rowsb, semb)

            prep(b, dstia)
            pltpu.make_async_copy(x_hbm.at[srcb.at[pl.ds(b * _G, _G)]],
                                  rowsa, sema).wait()
            pltpu.sync_copy(rowsa, acc.at[dstia], add=True)

            @pl.when(b + 1 < nbs)
            def _():
                @pl.when(b + 2 < nbs)
                def _():
                    fire(b + 2, rowsa, sema)

                prep(b + 1, dstib)
                pltpu.make_async_copy(x_hbm.at[srcb.at[pl.ds(b * _G, _G)]],
                                      rowsb, semb).wait()
                pltpu.sync_copy(rowsb, acc.at[dstib], add=True)

            return carry

        lax.fori_loop(0, lax.div(nbs + 1, 2), pipe, 0)
        plsc.subcore_barrier()
        pltpu.sync_copy(acc.at[pl.ds(s * rpt, rpt)],
                        out_hbm.at[c, pl.ds(s * rpt, rpt)])

    return k(x, src, dst, thr, zrows)


def _tc_finish(x, spair, cnt, w_lin, b_lin):
    n, d = x.shape
    blk = 400

    def body(x_ref, s0_ref, s1_ref, c_ref, w_ref, b_ref, o_ref):
        svec = s0_ref[0] + s1_ref[0]
        r = 0.5 / jnp.maximum(c_ref[...], 1.0)
        h = x_ref[...] + svec * r
        o_ref[...] = (
            jnp.dot(h, w_ref[...], preferred_element_type=jnp.float32)
            + b_ref[...]
        )

    return pl.pallas_call(
        body,
        grid=(n // blk,),
        in_specs=[
            pl.BlockSpec((blk, d), lambda i: (i, 0)),
            pl.BlockSpec((1, blk, d), lambda i: (0, i, 0)),
            pl.BlockSpec((1, blk, d), lambda i: (1, i, 0)),
            pl.BlockSpec((blk, 1), lambda i: (i, 0)),
            pl.BlockSpec((d, d), lambda i: (0, 0)),
            pl.BlockSpec((1, d), lambda i: (0, 0)),
        ],
        out_specs=pl.BlockSpec((blk, d), lambda i: (i, 0)),
        out_shape=jax.ShapeDtypeStruct((n, d), jnp.float32),
    )(x, spair, spair, cnt, w_lin, b_lin)


def kernel(x, edge_index, W_mlp, b_mlp, W_lin, b_lin):
    del W_mlp, b_mlp
    n, d = x.shape
    src = edge_index[0].astype(jnp.int32)
    dst = edge_index[1].astype(jnp.int32)

    thr, cnt = _sc_boundaries(dst, n)
    zrows = jnp.zeros((_npad(n) // _NS, d), jnp.float32)
    spair = _sc_aggregate(x, src, dst, thr, zrows)

    return _tc_finish(x, spair, cnt[:, None], W_lin, b_lin[None, :])


# speedup vs baseline: 40.3734x; 1.0607x over previous
import functools

import jax
import jax.numpy as jnp
from jax import lax
from jax.experimental import pallas as pl
from jax.experimental.pallas import tpu as pltpu
from jax.experimental.pallas import tpu_sc as plsc

_NC = 2
_NS = 16
_G = 64


def _npad(n):
    return -(-(n + 1) // 256) * 256


def _sc_boundaries(dst, n):
    e = dst.shape[0]
    npad = _npad(n)
    eb = e // _NS
    gb = eb // 16
    npt = npad // _NS
    mesh = plsc.VectorSubcoreMesh(core_axis_name="c", subcore_axis_name="s")

    @functools.partial(
        pl.kernel,
        out_type=(
            jax.ShapeDtypeStruct((npad,), jnp.int32),
            jax.ShapeDtypeStruct((npad,), jnp.float32),
        ),
        mesh=mesh,
        scratch_types=[
            pltpu.VMEM_SHARED((_NS, npad), jnp.int32),
            pltpu.VMEM((eb + 32,), jnp.int32),
            pltpu.VMEM((npad,), jnp.int32),
            pltpu.VMEM((npad,), jnp.int32),
            pltpu.VMEM((_NS, npt), jnp.int32),
            pltpu.VMEM((npt,), jnp.int32),
            pltpu.VMEM((npt,), jnp.float32),
        ],
        compiler_params=pltpu.CompilerParams(needs_layout_passes=False),
    )
    def k(dst_hbm, thr_hbm, cnt_hbm, st_sh, dstb2, stl, enl, mrg, thrt, cntt):
        c = lax.axis_index("c")
        s = lax.axis_index("s")
        iota = lax.iota(jnp.int32, 16)
        neg1 = jnp.full((16,), -1, jnp.int32)

        b0 = s * eb
        pltpu.sync_copy(dst_hbm.at[pl.ds(b0, eb)], dstb2.at[pl.ds(16, eb)])

        @pl.when(s == 0)
        def _():
            dstb2[pl.ds(0, 16)] = neg1

        @pl.when(s > 0)
        def _():
            pltpu.sync_copy(dst_hbm.at[pl.ds(b0 - 16, 16)],
                            dstb2.at[pl.ds(0, 16)])

        @pl.when(s == _NS - 1)
        def _():
            dstb2[pl.ds(16 + eb, 16)] = neg1

        @pl.when(s < _NS - 1)
        def _():
            pltpu.sync_copy(dst_hbm.at[pl.ds(b0 + eb, 16)],
                            dstb2.at[pl.ds(16 + eb, 16)])

        def init_body(i, carry):
            stl[pl.ds(i * 16, 16)] = neg1
            enl[pl.ds(i * 16, 16)] = neg1
            return carry

        lax.fori_loop(0, npad // 16, init_body, 0)

        def bound_body(g, carry):
            o = g * 16
            cur = dstb2[pl.ds(o + 16, 16)]
            prev = plsc.load_gather(dstb2, [o + 15 + iota])
            nxt = plsc.load_gather(dstb2, [o + 17 + iota])
            ev = b0 + o + iota
            plsc.store_scatter(stl, [cur], ev, mask=cur != prev)
            plsc.store_scatter(enl, [cur], ev + 1, mask=cur != nxt)
            return carry

        lax.fori_loop(0, gb, bound_body, 0)

        pltpu.sync_copy(stl, st_sh.at[s])
        plsc.subcore_barrier()
        pltpu.sync_copy(st_sh.at[:, pl.ds(s * npt, npt)], mrg)

        def merge_start(g, carry):
            m = neg1
            for t in range(_NS):
                m = jnp.maximum(m, mrg[t, pl.ds(g * 16, 16)])
            thrt[pl.ds(g * 16, 16)] = m
            return carry

        lax.fori_loop(0, npt // 16, merge_start, 0)
        plsc.subcore_barrier()
        pltpu.sync_copy(enl, st_sh.at[s])
        plsc.subcore_barrier()
        pltpu.sync_copy(st_sh.at[:, pl.ds(s * npt, npt)], mrg)

        def merge_fin(g, carry):
            m = neg1
            for t in range(_NS):
                m = jnp.maximum(m, mrg[t, pl.ds(g * 16, 16)])
            start = thrt[pl.ds(g * 16, 16)]
            q = lax.shift_right_arithmetic(m - start + 1, 1)
            thrt[pl.ds(g * 16, 16)] = start + q
            cntt[pl.ds(g * 16, 16)] = q.astype(jnp.float32)
            return carry

        lax.fori_loop(0, npt // 16, merge_fin, 0)

        @pl.when(c == 0)
        def _():
            pltpu.sync_copy(thrt, thr_hbm.at[pl.ds(s * npt, npt)])
            pltpu.sync_copy(cntt, cnt_hbm.at[pl.ds(s * npt, npt)])

    return k(dst)


def _sc_aggregate(x, src, dst, thr, zrows):
    n, d = x.shape
    e = src.shape[0]
    npad = thr.shape[0]
    nw = _NC * _NS
    ec = e // nw
    rpt = npad // _NS
    mesh = plsc.VectorSubcoreMesh(core_axis_name="c", subcore_axis_name="s")

    @functools.partial(
        pl.kernel,
        out_type=jax.ShapeDtypeStruct((_NC, npad, d), jnp.float32),
        mesh=mesh,
        scratch_types=[
            pltpu.VMEM_SHARED((npad, d), jnp.float32),
            pltpu.VMEM((ec + 16 + _G,), jnp.int32),
            pltpu.VMEM((ec + 16 + _G,), jnp.int32),
            pltpu.VMEM((npad,), jnp.int32),
            pltpu.VMEM((_G, d), jnp.float32),
            pltpu.VMEM((_G, d), jnp.float32),
            pltpu.VMEM((_G,), jnp.int32),
            pltpu.VMEM((_G,), jnp.int32),
            pltpu.SemaphoreType.DMA,
            pltpu.SemaphoreType.DMA,
        ],
        compiler_params=pltpu.CompilerParams(needs_layout_passes=False),
    )
    def k(x_hbm, src_hbm, dst_hbm, thr_hbm, z_hbm, out_hbm,
          acc, srcb, dstb, thrb, rowsa, rowsb, dstia, dstib, sema, semb):
        c = lax.axis_index("c")
        s = lax.axis_index("s")
        w = c * _NS + s
        iota = lax.iota(jnp.int32, 16)
        e0 = w * ec
        d0 = pltpu.async_copy(z_hbm, acc.at[pl.ds(s * rpt, rpt)], sema)
        d1 = pltpu.async_copy(src_hbm.at[pl.ds(e0, ec)],
                              srcb.at[pl.ds(0, ec)], sema)
        d2 = pltpu.async_copy(dst_hbm.at[pl.ds(e0, ec)],
                              dstb.at[pl.ds(0, ec)], sema)
        d3 = pltpu.async_copy(thr_hbm, thrb, sema)
        d0.wait()
        d1.wait()
        d2.wait()
        d3.wait()
        plsc.subcore_barrier()

        def compact(g, wptr):
            off = g * 16
            dv = dstb[pl.ds(off, 16)]
            tv = plsc.load_gather(thrb, [dv])
            sel = (e0 + off + iota) < tv
            plsc.store_compressed(srcb.at[pl.ds(wptr, 16)],
                                  srcb[pl.ds(off, 16)], mask=sel)
            plsc.store_compressed(dstb.at[pl.ds(wptr, 16)], dv, mask=sel)
            return wptr + jnp.sum(sel.astype(jnp.int32))

        nsel = lax.fori_loop(0, ec // 16, compact, jnp.int32(0))
        nbs = lax.div(nsel + (_G - 1), _G)
        for j in range(_G // 16 + 1):
            plsc.store_scatter(dstb, [nsel + j * 16 + iota],
                               jnp.full((16,), n, jnp.int32))

        def fire(b, rows, sem):
            return pltpu.async_copy(x_hbm.at[srcb.at[pl.ds(b * _G, _G)]],
                                    rows, sem)

        def prep(b, dsti):
            for j in range(_G // 16):
                dsti[pl.ds(j * 16, 16)] = dstb[pl.ds(b * _G + j * 16, 16)]

        @pl.when(nbs > 0)
        def _():
            fire(0, rowsa, sema)

        def pipe(i, carry):
            b = 2 * i

            @pl.when(b + 1 < nbs)
            def _():
                fire(b + 1, rowsb, semb)

            prep(b, dstia)
            pltpu.make_async_copy(x_hbm.at[srcb.at[pl.ds(b * _G, _G)]],
                                  rowsa, sema).wait()
            pltpu.sync_copy(rowsa, acc.at[dstia], add=True)

            @pl.when(b + 1 < nbs)
            def _():
                @pl.when(b + 2 < nbs)
                def _():
                    fire(b + 2, rowsa, sema)

                prep(b + 1, dstib)
                pltpu.make_async_copy(x_hbm.at[srcb.at[pl.ds(b * _G, _G)]],
                                      rowsb, semb).wait()
                pltpu.sync_copy(rowsb, acc.at[dstib], add=True)

            return carry

        lax.fori_loop(0, lax.div(nbs + 1, 2), pipe, 0)
        plsc.subcore_barrier()
        pltpu.sync_copy(acc.at[pl.ds(s * rpt, rpt)],
                        out_hbm.at[c, pl.ds(s * rpt, rpt)])

    return k(x, src, dst, thr, zrows)


def _tc_finish(x, spair, cnt, w_lin, b_lin):
    n, d = x.shape
    blk = 1000

    def body(x_ref, s0_ref, s1_ref, c_ref, w_ref, b_ref, o_ref):
        svec = s0_ref[0] + s1_ref[0]
        r = 0.5 / jnp.maximum(c_ref[...], 1.0)
        h = x_ref[...] + svec * r
        o_ref[...] = (
            jnp.dot(h, w_ref[...], preferred_element_type=jnp.float32)
            + b_ref[...]
        )

    return pl.pallas_call(
        body,
        grid=(n // blk,),
        in_specs=[
            pl.BlockSpec((blk, d), lambda i: (i, 0)),
            pl.BlockSpec((1, blk, d), lambda i: (0, i, 0)),
            pl.BlockSpec((1, blk, d), lambda i: (1, i, 0)),
            pl.BlockSpec((blk, 1), lambda i: (i, 0)),
            pl.BlockSpec((d, d), lambda i: (0, 0)),
            pl.BlockSpec((1, d), lambda i: (0, 0)),
        ],
        out_specs=pl.BlockSpec((blk, d), lambda i: (i, 0)),
        out_shape=jax.ShapeDtypeStruct((n, d), jnp.float32),
    )(x, spair, spair, cnt, w_lin, b_lin)


def kernel(x, edge_index, W_mlp, b_mlp, W_lin, b_lin):
    del W_mlp, b_mlp
    n, d = x.shape
    src = edge_index[0].astype(jnp.int32)
    dst = edge_index[1].astype(jnp.int32)

    thr, cnt = _sc_boundaries(dst, n)
    zrows = jnp.zeros((_npad(n) // _NS, d), jnp.float32)
    spair = _sc_aggregate(x, src, dst, thr, zrows)

    return _tc_finish(x, spair, cnt[:, None], W_lin, b_lin[None, :])


# speedup vs baseline: 40.5765x; 1.0050x over previous
import functools

import jax
import jax.numpy as jnp
from jax import lax
from jax.experimental import pallas as pl
from jax.experimental.pallas import tpu as pltpu
from jax.experimental.pallas import tpu_sc as plsc

_NC = 2
_NS = 16
_G = 64


def _npad(n):
    return -(-(n + 1) // 256) * 256


def _sc_boundaries(dst, n):
    e = dst.shape[0]
    npad = _npad(n)
    eb = e // _NS
    gb = eb // 16
    npt = npad // _NS
    mesh = plsc.VectorSubcoreMesh(core_axis_name="c", subcore_axis_name="s")

    @functools.partial(
        pl.kernel,
        out_type=(
            jax.ShapeDtypeStruct((npad,), jnp.int32),
            jax.ShapeDtypeStruct((npad,), jnp.float32),
        ),
        mesh=mesh,
        scratch_types=[
            pltpu.VMEM_SHARED((_NS, npad), jnp.int32),
            pltpu.VMEM((eb + 32,), jnp.int32),
            pltpu.VMEM((npad,), jnp.int32),
            pltpu.VMEM((npad,), jnp.int32),
            pltpu.VMEM((_NS, npt), jnp.int32),
            pltpu.VMEM((npt,), jnp.int32),
            pltpu.VMEM((npt,), jnp.float32),
        ],
        compiler_params=pltpu.CompilerParams(needs_layout_passes=False),
    )
    def k(dst_hbm, thr_hbm, cnt_hbm, st_sh, dstb2, stl, enl, mrg, thrt, cntt):
        c = lax.axis_index("c")
        s = lax.axis_index("s")
        iota = lax.iota(jnp.int32, 16)
        neg1 = jnp.full((16,), -1, jnp.int32)

        b0 = s * eb
        pltpu.sync_copy(dst_hbm.at[pl.ds(b0, eb)], dstb2.at[pl.ds(16, eb)])

        @pl.when(s == 0)
        def _():
            dstb2[pl.ds(0, 16)] = neg1

        @pl.when(s > 0)
        def _():
            pltpu.sync_copy(dst_hbm.at[pl.ds(b0 - 16, 16)],
                            dstb2.at[pl.ds(0, 16)])

        def init_body(i, carry):
            stl[pl.ds(i * 16, 16)] = neg1
            enl[pl.ds(i * 16, 16)] = neg1
            return carry

        lax.fori_loop(0, npad // 16, init_body, 0)

        def bound_body(g, carry):
            o = g * 16
            cur = dstb2[pl.ds(o + 16, 16)]
            prev = plsc.load_gather(dstb2, [o + 15 + iota])
            cmp = cur != prev
            ev = b0 + o + iota
            plsc.store_scatter(stl, [cur], ev, mask=cmp)
            plsc.store_scatter(enl, [prev], ev, mask=cmp & (prev >= 0))
            return carry

        lax.fori_loop(0, gb, bound_body, 0)

        @pl.when(s == _NS - 1)
        def _():
            lastv = dstb2[pl.ds(eb, 16)]
            plsc.store_scatter(enl, [lastv],
                               jnp.full((16,), e, jnp.int32),
                               mask=iota == 15)

        pltpu.sync_copy(stl, st_sh.at[s])
        plsc.subcore_barrier()
        pltpu.sync_copy(st_sh.at[:, pl.ds(s * npt, npt)], mrg)

        def merge_start(g, carry):
            m = neg1
            for t in range(_NS):
                m = jnp.maximum(m, mrg[t, pl.ds(g * 16, 16)])
            thrt[pl.ds(g * 16, 16)] = m
            return carry

        lax.fori_loop(0, npt // 16, merge_start, 0)
        plsc.subcore_barrier()
        pltpu.sync_copy(enl, st_sh.at[s])
        plsc.subcore_barrier()
        pltpu.sync_copy(st_sh.at[:, pl.ds(s * npt, npt)], mrg)

        def merge_fin(g, carry):
            m = neg1
            for t in range(_NS):
                m = jnp.maximum(m, mrg[t, pl.ds(g * 16, 16)])
            start = thrt[pl.ds(g * 16, 16)]
            q = lax.shift_right_arithmetic(m - start + 1, 1)
            thrt[pl.ds(g * 16, 16)] = start + q
            cntt[pl.ds(g * 16, 16)] = q.astype(jnp.float32)
            return carry

        lax.fori_loop(0, npt // 16, merge_fin, 0)

        @pl.when(c == 0)
        def _():
            pltpu.sync_copy(thrt, thr_hbm.at[pl.ds(s * npt, npt)])
            pltpu.sync_copy(cntt, cnt_hbm.at[pl.ds(s * npt, npt)])

    return k(dst)


def _sc_aggregate(x, src, dst, thr, zrows):
    n, d = x.shape
    e = src.shape[0]
    npad = thr.shape[0]
    nw = _NC * _NS
    ec = e // nw
    rpt = npad // _NS
    mesh = plsc.VectorSubcoreMesh(core_axis_name="c", subcore_axis_name="s")

    @functools.partial(
        pl.kernel,
        out_type=jax.ShapeDtypeStruct((_NC, npad, d), jnp.float32),
        mesh=mesh,
        scratch_types=[
            pltpu.VMEM_SHARED((npad, d), jnp.float32),
            pltpu.VMEM((ec + 16 + _G,), jnp.int32),
            pltpu.VMEM((ec + 16 + _G,), jnp.int32),
            pltpu.VMEM((npad,), jnp.int32),
            pltpu.VMEM((_G, d), jnp.float32),
            pltpu.VMEM((_G, d), jnp.float32),
            pltpu.VMEM((_G,), jnp.int32),
            pltpu.VMEM((_G,), jnp.int32),
            pltpu.SemaphoreType.DMA,
            pltpu.SemaphoreType.DMA,
        ],
        compiler_params=pltpu.CompilerParams(needs_layout_passes=False),
    )
    def k(x_hbm, src_hbm, dst_hbm, thr_hbm, z_hbm, out_hbm,
          acc, srcb, dstb, thrb, rowsa, rowsb, dstia, dstib, sema, semb):
        c = lax.axis_index("c")
        s = lax.axis_index("s")
        w = c * _NS + s
        iota = lax.iota(jnp.int32, 16)
        e0 = w * ec
        d0 = pltpu.async_copy(z_hbm, acc.at[pl.ds(s * rpt, rpt)], sema)
        d1 = pltpu.async_copy(src_hbm.at[pl.ds(e0, ec)],
                              srcb.at[pl.ds(0, ec)], sema)
        d2 = pltpu.async_copy(dst_hbm.at[pl.ds(e0, ec)],
                              dstb.at[pl.ds(0, ec)], sema)
        d3 = pltpu.async_copy(thr_hbm, thrb, sema)
        d0.wait()
        d1.wait()
        d2.wait()
        d3.wait()
        plsc.subcore_barrier()

        def compact(g, wptr):
            off = g * 16
            dv = dstb[pl.ds(off, 16)]
            tv = plsc.load_gather(thrb, [dv])
            sel = (e0 + off + iota) < tv
            plsc.store_compressed(srcb.at[pl.ds(wptr, 16)],
                                  srcb[pl.ds(off, 16)], mask=sel)
            plsc.store_compressed(dstb.at[pl.ds(wptr, 16)], dv, mask=sel)
            return wptr + jnp.sum(sel.astype(jnp.int32))

        nsel = lax.fori_loop(0, ec // 16, compact, jnp.int32(0))
        nbs = lax.div(nsel + (_G - 1), _G)
        for j in range(_G // 16 + 1):
            plsc.store_scatter(dstb, [nsel + j * 16 + iota],
                               jnp.full((16,), n, jnp.int32))

        def fire(b, rows, sem):
            return pltpu.async_copy(x_hbm.at[srcb.at[pl.ds(b * _G, _G)]],
                                    rows, sem)

        def prep(b, dsti):
            for j in range(_G // 16):
                dsti[pl.ds(j * 16, 16)] = dstb[pl.ds(b * _G + j * 16, 16)]

        @pl.when(nbs > 0)
        def _():
            fire(0, rowsa, sema)

        def pipe(i, carry):
            b = 2 * i

            @pl.when(b + 1 < nbs)
            def _():
                fire(b + 1, rowsb, semb)

            prep(b, dstia)
            pltpu.make_async_copy(x_hbm.at[srcb.at[pl.ds(b * _G, _G)]],
                                  rowsa, sema).wait()
            pltpu.sync_copy(rowsa, acc.at[dstia], add=True)

            @pl.when(b + 1 < nbs)
            def _():
                @pl.when(b + 2 < nbs)
                def _():
                    fire(b + 2, rowsa, sema)

                prep(b + 1, dstib)
                pltpu.make_async_copy(x_hbm.at[srcb.at[pl.ds(b * _G, _G)]],
                                      rowsb, semb).wait()
                pltpu.sync_copy(rowsb, acc.at[dstib], add=True)

            return carry

        lax.fori_loop(0, lax.div(nbs + 1, 2), pipe, 0)
        plsc.subcore_barrier()
        pltpu.sync_copy(acc.at[pl.ds(s * rpt, rpt)],
                        out_hbm.at[c, pl.ds(s * rpt, rpt)])

    return k(x, src, dst, thr, zrows)


def _tc_finish(x, spair, cnt, w_lin, b_lin):
    n, d = x.shape
    blk = 1000

    def body(x_ref, s0_ref, s1_ref, c_ref, w_ref, b_ref, o_ref):
        svec = s0_ref[0] + s1_ref[0]
        r = 0.5 / jnp.maximum(c_ref[...], 1.0)
        h = x_ref[...] + svec * r
        o_ref[...] = (
            jnp.dot(h, w_ref[...], preferred_element_type=jnp.float32)
            + b_ref[...]
        )

    return pl.pallas_call(
        body,
        grid=(n // blk,),
        in_specs=[
            pl.BlockSpec((blk, d), lambda i: (i, 0)),
            pl.BlockSpec((1, blk, d), lambda i: (0, i, 0)),
            pl.BlockSpec((1, blk, d), lambda i: (1, i, 0)),
            pl.BlockSpec((blk, 1), lambda i: (i, 0)),
            pl.BlockSpec((d, d), lambda i: (0, 0)),
            pl.BlockSpec((1, d), lambda i: (0, 0)),
        ],
        out_specs=pl.BlockSpec((blk, d), lambda i: (i, 0)),
        out_shape=jax.ShapeDtypeStruct((n, d), jnp.float32),
    )(x, spair, spair, cnt, w_lin, b_lin)


def kernel(x, edge_index, W_mlp, b_mlp, W_lin, b_lin):
    del W_mlp, b_mlp
    n, d = x.shape
    src = edge_index[0].astype(jnp.int32)
    dst = edge_index[1].astype(jnp.int32)

    thr, cnt = _sc_boundaries(dst, n)
    zrows = jnp.zeros((_npad(n) // _NS, d), jnp.float32)
    spair = _sc_aggregate(x, src, dst, thr, zrows)

    return _tc_finish(x, spair, cnt[:, None], W_lin, b_lin[None, :])
